# baseline (speedup 1.0000x reference)
import jax
import jax.numpy as jnp
import numpy as np
from jax.experimental import pallas as pl
from jax.experimental.pallas import tpu as pltpu

_N_TILES = 8
_L = 8
_NSQ = _N_TILES * _N_TILES
_OUT_COLS = _L * _NSQ

_REPL = np.repeat(np.eye(_L, dtype=np.float32), _NSQ, axis=1)
_BINID = (np.arange(_OUT_COLS, dtype=np.int64) % _NSQ).astype(np.float32)[None, :]


def _tyler_body(x_ref, t_ref, o_ref):
    x = x_ref[:, 0:1]
    y = x_ref[:, 1:2]
    tx = t_ref[0:1, :]
    ty = t_ref[1:2, :]
    dxt = jnp.float32(1.2)
    n = jnp.float32(_N_TILES)
    ix = ((x - tx) / dxt * n).astype(jnp.int32)
    iy = ((y - ty) / dxt * n).astype(jnp.int32)
    idx = ix + iy * _N_TILES
    idxb = idx.astype(jnp.bfloat16)
    lane = jax.lax.broadcasted_iota(jnp.int32, (_L, _OUT_COLS), 1)
    row = jax.lax.broadcasted_iota(jnp.int32, (_L, _OUT_COLS), 0)
    rep = ((lane >> 6) == row).astype(jnp.bfloat16)
    bcast = jax.lax.dot_general(
        idxb, rep, (((1,), (0,)), ((), ())),
        preferred_element_type=jnp.float32,
    )
    binid = (
        jax.lax.broadcasted_iota(jnp.int32, (1, _OUT_COLS), 1) & (_NSQ - 1)
    ).astype(jnp.float32)
    o_ref[:, :] = (bcast == binid).astype(jnp.float32)


import functools
from jax import lax
from jax.experimental.pallas import tpu_sc as plsc

_NW = 32
_R = 80


def _sc_body(xs_hbm, ys_hbm, tx_hbm, ty_hbm, out_hbm,
             xs_v, ys_v, tx_v, ty_v, buf):
    wid = lax.axis_index("s") * 2 + lax.axis_index("c")
    n_chunks = 50000 // _R
    pltpu.sync_copy(tx_hbm, tx_v)
    pltpu.sync_copy(ty_hbm, ty_v)
    zeros = jnp.zeros((16,), jnp.float32)

    def zloop(i, _):
        buf[pl.ds(i * 16, 16)] = zeros
        return 0

    lax.fori_loop(0, _R * 512 // 16, zloop, 0)

    lanes = lax.iota(jnp.int32, 16)
    ones = jnp.ones((16,), jnp.float32)
    dxt = jnp.float32(1.2)
    n = jnp.float32(_N_TILES)

    def scatter_chunk(val):
        for g in range(_R // 16):
            xv = xs_v[pl.ds(g * 16, 16)]
            yv = ys_v[pl.ds(g * 16, 16)]
            rbase = (lanes + g * 16) * _OUT_COLS
            for l in range(_L):
                ix = ((xv - tx_v[l]) / dxt * n).astype(jnp.int32)
                iy = ((yv - ty_v[l]) / dxt * n).astype(jnp.int32)
                idx = ix + iy * _N_TILES
                ok = (idx >= 0) & (idx < _NSQ)
                p = rbase + (l * _NSQ + idx)
                plsc.store_scatter(buf, [p], val, mask=ok)

    def body(i, _):
        c = wid + i * _NW
        off = c * _R
        pltpu.sync_copy(xs_hbm.at[pl.ds(off, _R)], xs_v)
        pltpu.sync_copy(ys_hbm.at[pl.ds(off, _R)], ys_v)
        scatter_chunk(ones)
        pltpu.sync_copy(buf, out_hbm.at[pl.ds(off * _OUT_COLS, _R * _OUT_COLS)])
        scatter_chunk(zeros)
        return 0

    my_chunks = (n_chunks - wid + _NW - 1) // _NW
    lax.fori_loop(0, my_chunks, body, 0)


@functools.partial(
    pl.kernel,
    mesh=plsc.VectorSubcoreMesh(core_axis_name="c", subcore_axis_name="s"),
    out_type=jax.ShapeDtypeStruct((50000 * _OUT_COLS,), jnp.float32),
    compiler_params=pltpu.CompilerParams(needs_layout_passes=False),
    scratch_types=[
        pltpu.VMEM((_R,), jnp.float32),
        pltpu.VMEM((_R,), jnp.float32),
        pltpu.VMEM((_L, 16), jnp.float32),
        pltpu.VMEM((_L, 16), jnp.float32),
        pltpu.VMEM((_R * _OUT_COLS,), jnp.float32),
    ],
)
def _sc_kernel(xs_hbm, ys_hbm, tx_hbm, ty_hbm, out_hbm,
               xs_v, ys_v, tx_v, ty_v, buf):
    _sc_body(xs_hbm, ys_hbm, tx_hbm, ty_hbm, out_hbm,
             xs_v, ys_v, tx_v, ty_v, buf)


@jax.jit
def kernel(x, tile0):
    xs = x[:, 0]
    ys = x[:, 1]
    tx = jnp.repeat(tile0[:, 0:1], 16, axis=1)
    ty = jnp.repeat(tile0[:, 1:2], 16, axis=1)
    flat = _sc_kernel(xs, ys, tx, ty)
    return flat.reshape(x.shape[0], _OUT_COLS)


@jax.jit
def _tc_kernel(x, tile0):
    n_points = x.shape[0]
    block = 5000
    grid = (pl.cdiv(n_points, block),)
    t_t = tile0.T
    return pl.pallas_call(
        _tyler_body,
        grid=grid,
        in_specs=[
            pl.BlockSpec((block, 2), lambda i: (i, 0)),
            pl.BlockSpec((2, _L), lambda i: (0, 0)),
        ],
        out_specs=pl.BlockSpec((block, _OUT_COLS), lambda i: (i, 0)),
        out_shape=jax.ShapeDtypeStruct((n_points, _OUT_COLS), jnp.float32),
        compiler_params=pltpu.CompilerParams(
            dimension_semantics=("parallel",),
        ),
    )(x, t_t)


# speedup vs baseline: 1.1536x; 1.1536x over previous
import jax
import jax.numpy as jnp
import numpy as np
from jax.experimental import pallas as pl
from jax.experimental.pallas import tpu as pltpu

_N_TILES = 8
_L = 8
_NSQ = _N_TILES * _N_TILES
_OUT_COLS = _L * _NSQ

_REPL = np.repeat(np.eye(_L, dtype=np.float32), _NSQ, axis=1)
_BINID = (np.arange(_OUT_COLS, dtype=np.int64) % _NSQ).astype(np.float32)[None, :]


def _tyler_body(x_ref, t_ref, o_ref):
    x = x_ref[:, 0:1]
    y = x_ref[:, 1:2]
    tx = t_ref[0:1, :]
    ty = t_ref[1:2, :]
    dxt = jnp.float32(1.2)
    n = jnp.float32(_N_TILES)
    ix = ((x - tx) / dxt * n).astype(jnp.int32)
    iy = ((y - ty) / dxt * n).astype(jnp.int32)
    idx = ix + iy * _N_TILES
    idxb = idx.astype(jnp.bfloat16)
    lane = jax.lax.broadcasted_iota(jnp.int32, (_L, _OUT_COLS), 1)
    row = jax.lax.broadcasted_iota(jnp.int32, (_L, _OUT_COLS), 0)
    rep = ((lane >> 6) == row).astype(jnp.bfloat16)
    bcast = jax.lax.dot_general(
        idxb, rep, (((1,), (0,)), ((), ())),
        preferred_element_type=jnp.float32,
    )
    binid = (
        jax.lax.broadcasted_iota(jnp.int32, (1, _OUT_COLS), 1) & (_NSQ - 1)
    ).astype(jnp.float32)
    o_ref[:, :] = (bcast == binid).astype(jnp.float32)


import functools
from jax import lax
from jax.experimental.pallas import tpu_sc as plsc

_NW = 32
_R = 80


_NCH = 50000 // _R
_MAXC = (_NCH + _NW - 1) // _NW
_XPAD = _MAXC * _R


def _sc_body(xs_hbm, ys_hbm, tx_hbm, ty_hbm, out_hbm,
             xs_v, ys_v, tx_v, ty_v, buf0, buf1, sem0, sem1):
    wid = lax.axis_index("s") * 2 + lax.axis_index("c")
    start = wid * (_MAXC - 1) + jnp.minimum(wid, _NCH - _NW * (_MAXC - 1))
    count = jnp.where(wid < _NCH - _NW * (_MAXC - 1), _MAXC, _MAXC - 1)
    pltpu.sync_copy(tx_hbm, tx_v)
    pltpu.sync_copy(ty_hbm, ty_v)
    pltpu.sync_copy(xs_hbm.at[pl.ds(start * _R, _XPAD)], xs_v)
    pltpu.sync_copy(ys_hbm.at[pl.ds(start * _R, _XPAD)], ys_v)
    zeros = jnp.zeros((16,), jnp.float32)

    bufs = (buf0, buf1)
    sems = (sem0, sem1)

    def zloop(i, _):
        buf0[pl.ds(i * 16, 16)] = zeros
        buf1[pl.ds(i * 16, 16)] = zeros
        return 0

    lax.fori_loop(0, _R * _OUT_COLS // 16, zloop, 0)

    lanes = lax.iota(jnp.int32, 16)
    ones = jnp.ones((16,), jnp.float32)
    dxt = jnp.float32(1.2)
    n = jnp.float32(_N_TILES)

    def scatter_chunk(buf, i, val):
        for g in range(_R // 16):
            xv = xs_v[pl.ds(i * _R + g * 16, 16)]
            yv = ys_v[pl.ds(i * _R + g * 16, 16)]
            rbase = (lanes + g * 16) * _OUT_COLS
            for l in range(_L):
                ix = ((xv - tx_v[l]) / dxt * n).astype(jnp.int32)
                iy = ((yv - ty_v[l]) / dxt * n).astype(jnp.int32)
                idx = ix + iy * _N_TILES
                ok = (idx >= 0) & (idx < _NSQ)
                p = rbase + (l * _NSQ + idx)
                plsc.store_scatter(buf, [p], val, mask=ok)

    def out_slice(i):
        return out_hbm.at[pl.ds((start + i) * _R * _OUT_COLS, _R * _OUT_COLS)]

    def pair_body(j, _):
        for b in range(2):
            i = 2 * j + b

            @pl.when(i < count)
            def _():
                @pl.when(i >= 2)
                def _():
                    pltpu.make_async_copy(bufs[b], out_slice(i - 2),
                                          sems[b]).wait()
                    scatter_chunk(bufs[b], i - 2, zeros)

                scatter_chunk(bufs[b], i, ones)
                pltpu.async_copy(bufs[b], out_slice(i), sems[b])
        return 0

    lax.fori_loop(0, (_MAXC + 1) // 2, pair_body, 0)
    pltpu.make_async_copy(buf0, out_slice(0), sem0).wait()
    pltpu.make_async_copy(buf1, out_slice(1), sem1).wait()


@functools.partial(
    pl.kernel,
    mesh=plsc.VectorSubcoreMesh(core_axis_name="c", subcore_axis_name="s"),
    out_type=jax.ShapeDtypeStruct((50000 * _OUT_COLS,), jnp.float32),
    compiler_params=pltpu.CompilerParams(needs_layout_passes=False),
    scratch_types=[
        pltpu.VMEM((_XPAD,), jnp.float32),
        pltpu.VMEM((_XPAD,), jnp.float32),
        pltpu.VMEM((_L, 16), jnp.float32),
        pltpu.VMEM((_L, 16), jnp.float32),
        pltpu.VMEM((_R * _OUT_COLS,), jnp.float32),
        pltpu.VMEM((_R * _OUT_COLS,), jnp.float32),
        pltpu.SemaphoreType.DMA,
        pltpu.SemaphoreType.DMA,
    ],
)
def _sc_kernel(xs_hbm, ys_hbm, tx_hbm, ty_hbm, out_hbm,
               xs_v, ys_v, tx_v, ty_v, buf0, buf1, sem0, sem1):
    _sc_body(xs_hbm, ys_hbm, tx_hbm, ty_hbm, out_hbm,
             xs_v, ys_v, tx_v, ty_v, buf0, buf1, sem0, sem1)


@jax.jit
def kernel(x, tile0):
    xs = jnp.pad(x[:, 0], (0, _XPAD))
    ys = jnp.pad(x[:, 1], (0, _XPAD))
    tx = jnp.repeat(tile0[:, 0:1], 16, axis=1)
    ty = jnp.repeat(tile0[:, 1:2], 16, axis=1)
    flat = _sc_kernel(xs, ys, tx, ty)
    return flat.reshape(x.shape[0], _OUT_COLS)


@jax.jit
def _tc_kernel(x, tile0):
    n_points = x.shape[0]
    block = 5000
    grid = (pl.cdiv(n_points, block),)
    t_t = tile0.T
    return pl.pallas_call(
        _tyler_body,
        grid=grid,
        in_specs=[
            pl.BlockSpec((block, 2), lambda i: (i, 0)),
            pl.BlockSpec((2, _L), lambda i: (0, 0)),
        ],
        out_specs=pl.BlockSpec((block, _OUT_COLS), lambda i: (i, 0)),
        out_shape=jax.ShapeDtypeStruct((n_points, _OUT_COLS), jnp.float32),
        compiler_params=pltpu.CompilerParams(
            dimension_semantics=("parallel",),
        ),
    )(x, t_t)


# speedup vs baseline: 3.8711x; 3.3556x over previous
import jax
import jax.numpy as jnp
import numpy as np
from jax.experimental import pallas as pl
from jax.experimental.pallas import tpu as pltpu

_N_TILES = 8
_L = 8
_NSQ = _N_TILES * _N_TILES
_OUT_COLS = _L * _NSQ

_REPL = np.repeat(np.eye(_L, dtype=np.float32), _NSQ, axis=1)
_BINID = (np.arange(_OUT_COLS, dtype=np.int64) % _NSQ).astype(np.float32)[None, :]


def _tyler_body(x_ref, t_ref, o_ref):
    x = x_ref[:, 0:1]
    y = x_ref[:, 1:2]
    tx = t_ref[0:1, :]
    ty = t_ref[1:2, :]
    dxt = jnp.float32(1.2)
    n = jnp.float32(_N_TILES)
    ix = ((x - tx) / dxt * n).astype(jnp.int32)
    iy = ((y - ty) / dxt * n).astype(jnp.int32)
    idx = ix + iy * _N_TILES
    idxb = idx.astype(jnp.bfloat16)
    lane = jax.lax.broadcasted_iota(jnp.int32, (_L, _OUT_COLS), 1)
    row = jax.lax.broadcasted_iota(jnp.int32, (_L, _OUT_COLS), 0)
    rep = ((lane >> 6) == row).astype(jnp.bfloat16)
    bcast = jax.lax.dot_general(
        idxb, rep, (((1,), (0,)), ((), ())),
        preferred_element_type=jnp.float32,
    )
    binid = (
        jax.lax.broadcasted_iota(jnp.int32, (1, _OUT_COLS), 1) & (_NSQ - 1)
    ).astype(jnp.float32)
    o_ref[:, :] = (bcast == binid).astype(jnp.float32)


import functools
from jax import lax
from jax.experimental.pallas import tpu_sc as plsc

_NW = 32
_R = 80


_NCH = 50000 // _R
_MAXC = (_NCH + _NW - 1) // _NW
_XPAD = _MAXC * _R


def _sc_body(xs_hbm, ys_hbm, tx_hbm, ty_hbm, out_hbm,
             xs_v, ys_v, tx_v, ty_v, buf0, buf1, sem0, sem1):
    wid = lax.axis_index("s") * 2 + lax.axis_index("c")
    start = wid * (_MAXC - 1) + jnp.minimum(wid, _NCH - _NW * (_MAXC - 1))
    count = jnp.where(wid < _NCH - _NW * (_MAXC - 1), _MAXC, _MAXC - 1)
    pltpu.sync_copy(tx_hbm, tx_v)
    pltpu.sync_copy(ty_hbm, ty_v)
    pltpu.sync_copy(xs_hbm.at[pl.ds(start * _R, _XPAD)], xs_v)
    pltpu.sync_copy(ys_hbm.at[pl.ds(start * _R, _XPAD)], ys_v)
    zeros = jnp.zeros((16,), jnp.float32)

    bufs = (buf0, buf1)
    sems = (sem0, sem1)

    def zloop(i, _):
        buf0[pl.ds(i * 16, 16)] = zeros
        buf1[pl.ds(i * 16, 16)] = zeros
        return 0

    lax.fori_loop(0, _R * _OUT_COLS // 16, zloop, 0)

    lanes = lax.iota(jnp.int32, 16)
    ones = jnp.ones((16,), jnp.float32)
    dxt = jnp.float32(1.2)
    n = jnp.float32(_N_TILES)

    def scatter_chunk(buf, i, val):
        for g in range(_R // 16):
            xv = xs_v[pl.ds(i * _R + g * 16, 16)]
            yv = ys_v[pl.ds(i * _R + g * 16, 16)]
            rbase = (lanes + g * 16) * _OUT_COLS
            for l in range(_L):
                ix = ((xv - tx_v[l]) / dxt * n).astype(jnp.int32)
                iy = ((yv - ty_v[l]) / dxt * n).astype(jnp.int32)
                idx = ix + iy * _N_TILES
                ok = (idx >= 0) & (idx < _NSQ)
                p = rbase + (l * _NSQ + idx)
                plsc.store_scatter(buf, [p], val, mask=ok)

    def out_slice(i):
        return out_hbm.at[pl.ds((start + i) * _R * _OUT_COLS, _R * _OUT_COLS)]

    def pair_body(j, _):
        for b in range(2):
            i = 2 * j + b

            @pl.when(i < count)
            def _():
                @pl.when(i >= 2)
                def _():
                    pltpu.make_async_copy(bufs[b], out_slice(i - 2),
                                          sems[b]).wait()
                    scatter_chunk(bufs[b], i - 2, zeros)

                scatter_chunk(bufs[b], i, ones)
                pltpu.async_copy(bufs[b], out_slice(i), sems[b])
        return 0

    lax.fori_loop(0, (_MAXC + 1) // 2, pair_body, 0)
    pltpu.make_async_copy(buf0, out_slice(0), sem0).wait()
    pltpu.make_async_copy(buf1, out_slice(1), sem1).wait()


@functools.partial(
    pl.kernel,
    mesh=plsc.VectorSubcoreMesh(core_axis_name="c", subcore_axis_name="s"),
    out_type=jax.ShapeDtypeStruct((50000 * _OUT_COLS,), jnp.float32),
    compiler_params=pltpu.CompilerParams(needs_layout_passes=False),
    scratch_types=[
        pltpu.VMEM((_XPAD,), jnp.float32),
        pltpu.VMEM((_XPAD,), jnp.float32),
        pltpu.VMEM((_L, 16), jnp.float32),
        pltpu.VMEM((_L, 16), jnp.float32),
        pltpu.VMEM((_R * _OUT_COLS,), jnp.float32),
        pltpu.VMEM((_R * _OUT_COLS,), jnp.float32),
        pltpu.SemaphoreType.DMA,
        pltpu.SemaphoreType.DMA,
    ],
)
def _sc_kernel(xs_hbm, ys_hbm, tx_hbm, ty_hbm, out_hbm,
               xs_v, ys_v, tx_v, ty_v, buf0, buf1, sem0, sem1):
    _sc_body(xs_hbm, ys_hbm, tx_hbm, ty_hbm, out_hbm,
             xs_v, ys_v, tx_v, ty_v, buf0, buf1, sem0, sem1)


@jax.jit
def _sc_entry(x, tile0):
    xs = jnp.pad(x[:, 0], (0, _XPAD))
    ys = jnp.pad(x[:, 1], (0, _XPAD))
    tx = jnp.repeat(tile0[:, 0:1], 16, axis=1)
    ty = jnp.repeat(tile0[:, 1:2], 16, axis=1)
    flat = _sc_kernel(xs, ys, tx, ty)
    return flat.reshape(x.shape[0], _OUT_COLS)


@jax.jit
def kernel(x, tile0):
    n_points = x.shape[0]
    block = 4000
    grid = (pl.cdiv(n_points, block),)
    t_t = tile0.T
    return pl.pallas_call(
        _tyler_body,
        grid=grid,
        in_specs=[
            pl.BlockSpec((block, 2), lambda i: (i, 0)),
            pl.BlockSpec((2, _L), lambda i: (0, 0)),
        ],
        out_specs=pl.BlockSpec((block, _OUT_COLS), lambda i: (i, 0)),
        out_shape=jax.ShapeDtypeStruct((n_points, _OUT_COLS), jnp.float32),
        compiler_params=pltpu.CompilerParams(
            dimension_semantics=("parallel",),
        ),
    )(x, t_t)


# speedup vs baseline: 3.8921x; 1.0054x over previous
import jax
import jax.numpy as jnp
from jax.experimental import pallas as pl
from jax.experimental.pallas import tpu as pltpu

_N_TILES = 8
_L = 8
_NSQ = _N_TILES * _N_TILES
_OUT_COLS = _L * _NSQ


def _tyler_body(x_ref, t_ref, o_ref):
    x = x_ref[:, 0:1]
    y = x_ref[:, 1:2]
    tx = t_ref[0:1, :]
    ty = t_ref[1:2, :]
    dxt = jnp.float32(1.2)
    n = jnp.float32(_N_TILES)
    ix = ((x - tx) / dxt * n).astype(jnp.int32)
    iy = ((y - ty) / dxt * n).astype(jnp.int32)
    idx = ix + iy * _N_TILES
    idxb = idx.astype(jnp.bfloat16)
    lane = jax.lax.broadcasted_iota(jnp.int32, (_L, _OUT_COLS), 1)
    row = jax.lax.broadcasted_iota(jnp.int32, (_L, _OUT_COLS), 0)
    rep = ((lane >> 6) == row).astype(jnp.bfloat16)
    bcast = jax.lax.dot_general(
        idxb, rep, (((1,), (0,)), ((), ())),
        preferred_element_type=jnp.float32,
    )
    binid = (
        jax.lax.broadcasted_iota(jnp.int32, (1, _OUT_COLS), 1) & (_NSQ - 1)
    ).astype(jnp.float32)
    o_ref[:, :] = (bcast == binid).astype(jnp.float32)


@jax.jit
def kernel(x, tile0):
    n_points = x.shape[0]
    block = 5000
    grid = (pl.cdiv(n_points, block),)
    t_t = tile0.T
    return pl.pallas_call(
        _tyler_body,
        grid=grid,
        in_specs=[
            pl.BlockSpec((block, 2), lambda i: (i, 0)),
            pl.BlockSpec((2, _L), lambda i: (0, 0)),
        ],
        out_specs=pl.BlockSpec((block, _OUT_COLS), lambda i: (i, 0)),
        out_shape=jax.ShapeDtypeStruct((n_points, _OUT_COLS), jnp.float32),
        compiler_params=pltpu.CompilerParams(
            dimension_semantics=("parallel",),
        ),
    )(x, t_t)
